# baseline (speedup 1.0000x reference)
import jax
import jax.numpy as jnp
from jax.experimental import pallas as pl
from jax.experimental.pallas import tpu as pltpu

_NCHUNKS = 4


def _dma_body(t_hbm, o_hbm, buf, in_sems, out_sems):
    rows = o_hbm.shape[0]
    chunk = rows // _NCHUNKS

    def copy_in(i, slot):
        return pltpu.make_async_copy(
            t_hbm.at[pl.ds(i * chunk, chunk), :], buf.at[slot], in_sems.at[slot]
        )

    def copy_out(i, slot):
        return pltpu.make_async_copy(
            buf.at[slot], o_hbm.at[pl.ds(i * chunk, chunk), :], out_sems.at[slot]
        )

    copy_in(0, 0).start()
    for i in range(_NCHUNKS):
        slot = i % 2
        copy_in(i, slot).wait()
        if i + 1 < _NCHUNKS:
            if i >= 1:
                copy_out(i - 1, (i + 1) % 2).wait()
            copy_in(i + 1, (i + 1) % 2).start()
        copy_out(i, slot).start()
    copy_out(_NCHUNKS - 1, (_NCHUNKS - 1) % 2).wait()
    copy_out(_NCHUNKS - 2, (_NCHUNKS - 2) % 2).wait()


def kernel(x, table):
    seq_len = x.shape[1]
    dim = table.shape[1]
    chunk = seq_len // _NCHUNKS
    return pl.pallas_call(
        _dma_body,
        in_specs=[pl.BlockSpec(memory_space=pl.ANY)],
        out_specs=pl.BlockSpec(memory_space=pl.ANY),
        out_shape=jax.ShapeDtypeStruct((seq_len, dim), table.dtype),
        scratch_shapes=[
            pltpu.VMEM((2, chunk, dim), table.dtype),
            pltpu.SemaphoreType.DMA((2,)),
            pltpu.SemaphoreType.DMA((2,)),
        ],
    )(table)


# speedup vs baseline: 1.2372x; 1.2372x over previous
import jax
import jax.numpy as jnp
from jax.experimental import pallas as pl

_BLOCK_ROWS = 4096


def _copy_body(t_ref, o_ref):
    o_ref[...] = t_ref[...]


def kernel(x, table):
    seq_len = x.shape[1]
    dim = table.shape[1]
    return pl.pallas_call(
        _copy_body,
        grid=(seq_len // _BLOCK_ROWS,),
        in_specs=[pl.BlockSpec((_BLOCK_ROWS, dim), lambda i: (i, 0))],
        out_specs=pl.BlockSpec((_BLOCK_ROWS, dim), lambda i: (i, 0)),
        out_shape=jax.ShapeDtypeStruct((seq_len, dim), table.dtype),
    )(table)
